# baseline (speedup 1.0000x reference)
import functools

import jax
import jax.numpy as jnp
from jax import lax
from jax.experimental import pallas as pl
from jax.experimental.pallas import tpu as pltpu
from jax.experimental.pallas import tpu_sc as plsc

L = 16


def kernel(dom_idx, paras):
  B = dom_idx.shape[0]
  V, D = paras.shape
  info = plsc.get_sparse_core_info()
  nc = info.num_cores
  nw = nc * info.num_subcores
  b_per_w = B // nw
  n_groups = b_per_w // L

  mesh = plsc.VectorSubcoreMesh(core_axis_name="c", subcore_axis_name="s")

  @functools.partial(
      pl.kernel,
      mesh=mesh,
      out_type=jax.ShapeDtypeStruct((D, B), jnp.float32),
      compiler_params=pltpu.CompilerParams(needs_layout_passes=False),
      scratch_types=[
          pltpu.VMEM((b_per_w,), jnp.int32),
          pltpu.VMEM((b_per_w, D), jnp.float32),
          pltpu.VMEM((D, b_per_w), jnp.float32),
          pltpu.SemaphoreType.DMA,
      ],
  )
  def gather_kernel(idx_hbm, table_hbm, out_hbm, idx_v, rows_v, out_v, sem):
    wid = lax.axis_index("s") * nc + lax.axis_index("c")
    base = wid * b_per_w
    pltpu.sync_copy(idx_hbm.at[pl.ds(base, b_per_w)], idx_v)
    iota = lax.iota(jnp.int32, L)

    def issue_group(g, carry):
      v = idx_v[pl.ds(g * L, L)]
      for l in range(L):
        s = lax.reduce_max(jnp.where(iota == l, v, 0), axes=(0,))
        pltpu.async_copy(
            table_hbm.at[pl.ds(s, 1)],
            rows_v.at[pl.ds(g * L + l, 1)],
            sem,
        )
      return carry

    lax.fori_loop(0, n_groups, issue_group, 0)
    pltpu.make_async_copy(
        table_hbm.at[pl.ds(0, b_per_w)], rows_v, sem
    ).wait()

    def transpose_group(g, carry):
      rowv = iota + g * L
      for c in range(D):
        val = plsc.load_gather(rows_v, [rowv, jnp.full((L,), c, jnp.int32)])
        out_v[c, pl.ds(g * L, L)] = val
      return carry

    lax.fori_loop(0, n_groups, transpose_group, 0)
    pltpu.sync_copy(out_v, out_hbm.at[:, pl.ds(base, b_per_w)])

  out_t = gather_kernel(dom_idx, paras)
  return out_t.T


# speedup vs baseline: 1.6107x; 1.6107x over previous
import functools

import jax
import jax.numpy as jnp
from jax import lax
from jax.experimental import pallas as pl
from jax.experimental.pallas import tpu as pltpu
from jax.experimental.pallas import tpu_sc as plsc

L = 16
BCHUNK = 4096


def kernel(dom_idx, paras):
  B = dom_idx.shape[0]
  V, D = paras.shape
  table_t = paras.T
  info = plsc.get_sparse_core_info()
  nc = info.num_cores
  nw = nc * info.num_subcores
  c_per_w = D // nw
  n_chunks = B // BCHUNK
  n_groups = BCHUNK // L

  mesh = plsc.VectorSubcoreMesh(core_axis_name="c", subcore_axis_name="s")

  @functools.partial(
      pl.kernel,
      mesh=mesh,
      out_type=jax.ShapeDtypeStruct((D, B), jnp.float32),
      compiler_params=pltpu.CompilerParams(needs_layout_passes=False),
      scratch_types=[
          pltpu.VMEM((B,), jnp.int32),
          pltpu.VMEM((1, V), jnp.float32),
          pltpu.VMEM((1, BCHUNK), jnp.float32),
          pltpu.SemaphoreType.DMA,
      ],
  )
  def gather_kernel(idx_hbm, table_hbm, out_hbm, idx_v, chan_v, outbuf_v,
                    sem):
    wid = lax.axis_index("s") * nc + lax.axis_index("c")
    pltpu.sync_copy(idx_hbm.at[...], idx_v)
    zeros = jnp.zeros((L,), jnp.int32)

    for j in range(c_per_w):
      ch = wid * c_per_w + j
      pltpu.sync_copy(table_hbm.at[pl.ds(ch, 1), :], chan_v)
      for k in range(n_chunks):

        def gather_group(g, carry, k=k):
          v = idx_v[pl.ds(k * BCHUNK + g * L, L)]
          outbuf_v[0, pl.ds(g * L, L)] = plsc.load_gather(
              chan_v, [zeros, v])
          return carry

        lax.fori_loop(0, n_groups, gather_group, 0)
        pltpu.sync_copy(
            outbuf_v, out_hbm.at[pl.ds(ch, 1), pl.ds(k * BCHUNK, BCHUNK)]
        )

  out_t = gather_kernel(dom_idx, table_t)
  return out_t.T


# speedup vs baseline: 1.8973x; 1.1780x over previous
import functools

import jax
import jax.numpy as jnp
from jax import lax
from jax.experimental import pallas as pl
from jax.experimental.pallas import tpu as pltpu
from jax.experimental.pallas import tpu_sc as plsc

L = 16
BCHUNK = 4096


def kernel(dom_idx, paras):
  B = dom_idx.shape[0]
  V, D = paras.shape
  table_t = paras.T
  info = plsc.get_sparse_core_info()
  nc = info.num_cores
  nw = nc * info.num_subcores
  c_per_w = D // nw
  n_chunks = B // BCHUNK
  n_groups = BCHUNK // L

  mesh = plsc.VectorSubcoreMesh(core_axis_name="c", subcore_axis_name="s")

  @functools.partial(
      pl.kernel,
      mesh=mesh,
      out_type=jax.ShapeDtypeStruct((D, B), jnp.float32),
      compiler_params=pltpu.CompilerParams(needs_layout_passes=False),
      scratch_types=[
          pltpu.VMEM((B,), jnp.int32),
          pltpu.VMEM((1, V), jnp.float32),
          pltpu.VMEM((1, BCHUNK), jnp.float32),
          pltpu.SemaphoreType.DMA,
      ],
  )
  def gather_kernel(idx_hbm, table_hbm, out_hbm, idx_v, chan_v, outbuf_v,
                    sem):
    wid = lax.axis_index("s") * nc + lax.axis_index("c")
    pltpu.sync_copy(idx_hbm.at[...], idx_v)
    zeros = jnp.zeros((L,), jnp.int32)

    for j in range(c_per_w):
      ch = wid * c_per_w + j
      pltpu.sync_copy(table_hbm.at[pl.ds(ch, 1), :], chan_v)
      for k in range(n_chunks):

        def gather_group(g8, carry, k=k):
          for u in range(8):
            g = g8 * 8 + u
            v = idx_v[pl.ds(k * BCHUNK + g * L, L)]
            outbuf_v[0, pl.ds(g * L, L)] = plsc.load_gather(
                chan_v, [zeros, v])
          return carry

        lax.fori_loop(0, n_groups // 8, gather_group, 0)
        pltpu.sync_copy(
            outbuf_v, out_hbm.at[pl.ds(ch, 1), pl.ds(k * BCHUNK, BCHUNK)]
        )

  out_t = gather_kernel(dom_idx, table_t)
  return out_t.T
